# baseline (speedup 1.0000x reference)
import functools

import jax
import jax.numpy as jnp
from jax import lax
from jax.experimental import pallas as pl
from jax.experimental.pallas import tpu as pltpu
from jax.experimental.pallas import tpu_sc as plsc

_N = 10000
_E = 160000
_D = 256
_DH = 128
_DHID = 512
_NC = 2
_NS = 16
_CH = 128
_GRP = 8
_NGRP = 10
_CPT = _GRP * _NGRP
_NCHK = _NS * _CPT
_EPAD = _NCHK * _CH
_NPAD = 10112
_TRASH = 10000
_AROWS = _NPAD // _NS
_NPC = 10240
_CSTR = _NPC // _NS
_CCT = _NCHK // _NC // _NS

_mesh = plsc.VectorSubcoreMesh(
    core_axis_name="c", subcore_axis_name="s", num_cores=_NC, num_subcores=_NS
)

_PIECES = [(0, 128), (128, 128), (256, 128), (384, 128), (512, 120)]


def _sc_agg_body(x2, src2, dst2, agg_o,
                 gidx_v, dst_v, rows_v,
                 acc_sh, gsem0, gsem1, isem):
    c = lax.axis_index("c")
    s = lax.axis_index("s")
    zero16 = jnp.zeros((16,), jnp.float32)

    def _zr(i, _):
        def _zc(j, _):
            rows_v[0, i, pl.ds(j * 16, 16)] = zero16
            return 0

        lax.fori_loop(0, _DH // 16, _zc, 0)
        return 0

    lax.fori_loop(0, _CH, _zr, 0)

    for r0, pr in _PIECES:
        pltpu.sync_copy(
            rows_v.at[0, pl.ds(0, pr)],
            acc_sh.at[pl.ds(s * _AROWS + r0, pr)],
        )

    plsc.subcore_barrier()

    gsems = (gsem0, gsem1)

    def _load_group(k, slot):
        r0 = s * _CPT + k * _GRP
        pltpu.async_copy(src2.at[pl.ds(r0, _GRP)], gidx_v.at[slot], isem)
        pltpu.async_copy(dst2.at[pl.ds(r0, _GRP)], dst_v.at[slot], isem)

    def _wait_group(slot):
        pltpu.make_async_copy(
            src2.at[pl.ds(0, _GRP)], gidx_v.at[slot], isem).wait()
        pltpu.make_async_copy(
            dst2.at[pl.ds(0, _GRP)], dst_v.at[slot], isem).wait()

    def _xform_group(slot):
        for u in range(_GRP):
            for j in range(_CH // 16):
                sl = pl.ds(j * 16, 16)
                gidx_v[slot, u, sl] = gidx_v[slot, u, sl] * 2 + c

    def _issue(gslot, u, rslot):
        pltpu.async_copy(
            x2.at[gidx_v.at[gslot, u]], rows_v.at[rslot], gsems[rslot])

    def _waitg(rslot):
        pltpu.make_async_copy(
            x2.at[gidx_v.at[0, 0]], rows_v.at[rslot], gsems[rslot]).wait()

    def _group(k, kslot, nslot, last):
        if not last:
            _load_group(k + 1, nslot)
        for u in range(_GRP):
            rslot = u % 2
            if u < _GRP - 1:
                _issue(kslot, u + 1, (u + 1) % 2)
            elif not last:
                _wait_group(nslot)
                _xform_group(nslot)
                _issue(nslot, 0, 0)
            _waitg(rslot)
            pltpu.sync_copy(
                rows_v.at[rslot], acc_sh.at[dst_v.at[kslot, u]], add=True)

    _load_group(0, 0)
    _wait_group(0)
    _xform_group(0)
    _issue(0, 0, 0)

    def _pairbody(t, _):
        _group(t * 2, 0, 1, False)
        _group(t * 2 + 1, 1, 0, False)
        return 0

    lax.fori_loop(0, _NGRP // 2 - 1, _pairbody, 0)
    _group(_NGRP - 2, 0, 1, False)
    _group(_NGRP - 1, 1, 0, True)

    plsc.subcore_barrier()

    for r0, pr in _PIECES:
        a0 = s * _AROWS + r0
        pltpu.sync_copy(acc_sh.at[pl.ds(a0, pr)], rows_v.at[0, pl.ds(0, pr)])
        pltpu.sync_copy(rows_v.at[0, pl.ds(0, pr)], agg_o.at[c, pl.ds(a0, pr)])


_sc_agg = functools.partial(
    pl.kernel,
    out_type=jax.ShapeDtypeStruct((_NC, _NPAD, _DH), jnp.float32),
    mesh=_mesh,
    scratch_types=[
        pltpu.VMEM((2, _GRP, _CH), jnp.int32),
        pltpu.VMEM((2, _GRP, _CH), jnp.int32),
        pltpu.VMEM((2, _CH, _DH), jnp.float32),
        pltpu.VMEM_SHARED((_NPAD, _DH), jnp.float32),
        pltpu.SemaphoreType.DMA,
        pltpu.SemaphoreType.DMA,
        pltpu.SemaphoreType.DMA,
    ],
    compiler_params=pltpu.CompilerParams(needs_layout_passes=False),
)(_sc_agg_body)


def _sc_cnt_body(dst2, cnt_o, dbuf_v, hist_v, hbuf_v, cbuf_v, cpart_sh):
    c = lax.axis_index("c")
    s = lax.axis_index("s")
    zero16 = jnp.zeros((16,), jnp.float32)
    one16 = jnp.ones((16,), jnp.float32)

    def _zh(i, _):
        hist_v[pl.ds(i * 16, 16)] = zero16
        return 0

    lax.fori_loop(0, _NPC // 16, _zh, 0)

    base = (c * _NS + s) * _CCT
    for k in range(_CCT // _GRP):
        pltpu.sync_copy(dst2.at[pl.ds(base + k * _GRP, _GRP)], dbuf_v)
        for r in range(_GRP):
            for j in range(_CH // 16):
                d16 = dbuf_v[r, pl.ds(j * 16, 16)]
                plsc.addupdate_scatter(hist_v, [d16], one16)

    pltpu.sync_copy(hist_v, cpart_sh.at[s])
    plsc.subcore_barrier()

    pltpu.sync_copy(cpart_sh.at[:, pl.ds(s * _CSTR, _CSTR)], hbuf_v)

    def _red(j, _):
        sl = pl.ds(j * 16, 16)
        a = hbuf_v[0, sl]
        for t in range(1, _NS):
            a = a + hbuf_v[t, sl]
        cbuf_v[sl] = a
        return 0

    lax.fori_loop(0, _CSTR // 16, _red, 0)
    pltpu.sync_copy(cbuf_v, cnt_o.at[c, pl.ds(s * _CSTR, _CSTR)])


_sc_cnt = functools.partial(
    pl.kernel,
    out_type=jax.ShapeDtypeStruct((_NC, _NPC), jnp.float32),
    mesh=_mesh,
    scratch_types=[
        pltpu.VMEM((_GRP, _CH), jnp.int32),
        pltpu.VMEM((_NPC,), jnp.float32),
        pltpu.VMEM((_NS, _CSTR), jnp.float32),
        pltpu.VMEM((_CSTR,), jnp.float32),
        pltpu.VMEM_SHARED((_NS, _NPC), jnp.float32),
    ],
    compiler_params=pltpu.CompilerParams(needs_layout_passes=False),
)(_sc_cnt_body)


_BN = 1000


def _tc_dense_body(agg_ref, cnt_ref, x_ref, wn_ref, bn_ref, ws_ref,
                   w1_ref, b1_ref, w2_ref, b2_ref, o_ref):
    a0 = agg_ref[0]
    a1 = agg_ref[1]
    recip = 1.0 / jnp.maximum(cnt_ref[...], 1.0)
    xb = x_ref[...]
    wn = wn_ref[...]
    conv = (
        jnp.dot(a0 * recip, wn[:_DH], preferred_element_type=jnp.float32)
        + jnp.dot(a1 * recip, wn[_DH:], preferred_element_type=jnp.float32)
        + jnp.dot(xb, ws_ref[...], preferred_element_type=jnp.float32)
        + bn_ref[...]
    )
    h = jnp.maximum(conv, 0.0)
    z = xb + h
    hid = jnp.maximum(
        jnp.dot(z, w1_ref[...], preferred_element_type=jnp.float32) + b1_ref[...],
        0.0,
    )
    o_ref[...] = h + jnp.dot(hid, w2_ref[...], preferred_element_type=jnp.float32) + b2_ref[...]


def _tc_dense(agg, cnt, x, wn, bn, ws, w1, b1, w2, b2):
    return pl.pallas_call(
        _tc_dense_body,
        grid=(_N // _BN,),
        in_specs=[
            pl.BlockSpec((_NC, _BN, _DH), lambda i: (0, i, 0)),
            pl.BlockSpec((_BN, 1), lambda i: (i, 0)),
            pl.BlockSpec((_BN, _D), lambda i: (i, 0)),
            pl.BlockSpec((_D, _D), lambda i: (0, 0)),
            pl.BlockSpec((1, _D), lambda i: (0, 0)),
            pl.BlockSpec((_D, _D), lambda i: (0, 0)),
            pl.BlockSpec((_D, _DHID), lambda i: (0, 0)),
            pl.BlockSpec((1, _DHID), lambda i: (0, 0)),
            pl.BlockSpec((_DHID, _D), lambda i: (0, 0)),
            pl.BlockSpec((1, _D), lambda i: (0, 0)),
        ],
        out_specs=pl.BlockSpec((_BN, _D), lambda i: (i, 0)),
        out_shape=jax.ShapeDtypeStruct((_N, _D), jnp.float32),
    )(agg, cnt, x, wn, bn, ws, w1, b1, w2, b2)


def kernel(x, edge_index, W_neigh, b_neigh, W_self, W1, b1, W2, b2):
    src = edge_index[0].astype(jnp.int32)
    dst = edge_index[1].astype(jnp.int32)
    npad = _EPAD - _E
    src2 = jnp.concatenate(
        [src, jnp.zeros((npad,), jnp.int32)]).reshape(_NCHK, _CH)
    dst2 = jnp.concatenate(
        [dst, jnp.full((npad,), _TRASH, jnp.int32)]).reshape(_NCHK, _CH)
    x2 = x.reshape(2 * _N, _DH)
    agg_pad = _sc_agg(x2, src2, dst2)
    cnt_pad = _sc_cnt(dst2)
    agg = agg_pad[:, :_N, :]
    cnt = (cnt_pad[0, :_N] + cnt_pad[1, :_N]).reshape(_N, 1)
    return _tc_dense(
        agg, cnt, x, W_neigh, b_neigh.reshape(1, _D), W_self,
        W1, b1.reshape(1, _DHID), W2, b2.reshape(1, _D),
    )
